# baseline (speedup 1.0000x reference)
import jax
import jax.numpy as jnp
from jax.experimental import pallas as pl
from jax.experimental.pallas import tpu as pltpu

_R = 256


def _vq_round_body(res_ref, sumz_ref, cb_ref, sume_ref,
                   codes_ref, zq_ref, newres_ref, d2_ref):
    t = pl.program_id(0)

    @pl.when(t == 0)
    def _init():
        d2_ref[...] = jnp.zeros_like(d2_ref)

    r = res_ref[...]
    emb = cb_ref[...]
    sumz = sumz_ref[...]
    sume = sume_ref[...]
    K = emb.shape[0]

    mm = jax.lax.dot_general(r, emb, (((1,), (1,)), ((), ())))
    dist = sumz - 2.0 * mm + sume
    m = jnp.min(dist, axis=1, keepdims=True)
    iota = jax.lax.broadcasted_iota(jnp.int32, (_R, K), 1)
    idx = jnp.min(jnp.where(dist == m, iota, K), axis=1,
                  keepdims=True)
    onehot = (iota == idx).astype(jnp.float32)
    zq_raw = jax.lax.dot_general(
        onehot, emb, (((1,), (0,)), ((), ())),
        precision=jax.lax.Precision.HIGHEST)
    d = zq_raw - r
    zq_st = r + d
    codes_ref[...] = idx
    zq_ref[...] = zq_st
    newres_ref[...] = r - zq_st
    d2_ref[...] = d2_ref[...] + jnp.sum(d * d, axis=(0, 1), keepdims=True)


def _mlp_body(x_ref, t_ref, w1_ref, b1_ref, w2_ref, b2_ref, out_ref):
    i = pl.program_id(0)

    @pl.when(i == 0)
    def _init():
        out_ref[...] = jnp.zeros_like(out_ref)

    x = x_ref[...]
    h = jnp.dot(x, w1_ref[...]) + b1_ref[...]
    h = 0.5 * h * (1.0 + jax.lax.erf(h * jnp.float32(0.7071067811865476)))
    pred = jnp.dot(h, w2_ref[...]) + b2_ref[...]
    e = pred - t_ref[...]
    out_ref[...] = out_ref[...] + jnp.sum(e * e, axis=(0, 1), keepdims=True)


def kernel(z, w2v_targets, codebooks, W1, b1, W2, b2):
    B, T, D = z.shape
    NCB, K, _ = codebooks.shape
    N = B * T
    z2 = z.reshape(N, D)
    sume_all = jnp.sum(codebooks ** 2, axis=2)

    vq_round = pl.pallas_call(
        _vq_round_body,
        grid=(N // _R,),
        in_specs=[
            pl.BlockSpec((_R, D), lambda t: (t, 0)),
            pl.BlockSpec((_R, 1), lambda t: (t, 0)),
            pl.BlockSpec((K, D), lambda t: (0, 0)),
            pl.BlockSpec((1, K), lambda t: (0, 0)),
        ],
        out_specs=[
            pl.BlockSpec((_R, 1), lambda t: (t, 0)),
            pl.BlockSpec((_R, D), lambda t: (t, 0)),
            pl.BlockSpec((_R, D), lambda t: (t, 0)),
            pl.BlockSpec((1, 1), lambda t: (0, 0)),
        ],
        out_shape=[
            jax.ShapeDtypeStruct((N, 1), jnp.int32),
            jax.ShapeDtypeStruct((N, D), jnp.float32),
            jax.ShapeDtypeStruct((N, D), jnp.float32),
            jax.ShapeDtypeStruct((1, 1), jnp.float32),
        ],
    )

    residual = z2
    zq_total = None
    zq0 = None
    codes = []
    vqsum = jnp.float32(0.0)
    for i in range(NCB):
        sumz = jnp.sum(residual ** 2, axis=1, keepdims=True)
        idx, zq_st, residual, d2 = vq_round(
            residual, sumz, codebooks[i], sume_all[i].reshape(1, K))
        codes.append(idx)
        zq_total = zq_st if i == 0 else zq_total + zq_st
        if i == 0:
            zq0 = zq_st
        vqsum = vqsum + d2[0, 0]

    RT = 1024
    F = w2v_targets.shape[-1]
    sse = pl.pallas_call(
        _mlp_body,
        grid=(N // RT,),
        in_specs=[
            pl.BlockSpec((RT, D), lambda i: (i, 0)),
            pl.BlockSpec((RT, F), lambda i: (i, 0)),
            pl.BlockSpec(W1.shape, lambda i: (0, 0)),
            pl.BlockSpec((1, W1.shape[1]), lambda i: (0, 0)),
            pl.BlockSpec(W2.shape, lambda i: (0, 0)),
            pl.BlockSpec((1, F), lambda i: (0, 0)),
        ],
        out_specs=pl.BlockSpec((1, 1), lambda i: (0, 0)),
        out_shape=jax.ShapeDtypeStruct((1, 1), jnp.float32),
    )(zq0, w2v_targets.reshape(N, F), W1, b1.reshape(1, -1), W2,
      b2.reshape(1, -1))

    all_codes = jnp.concatenate(codes, axis=1).reshape(B, T, NCB)
    vq_loss = vqsum * jnp.float32(1.25 / (N * D))
    semantic_loss = sse[0, 0] / jnp.float32(N * F)
    return (zq_total.reshape(B, T, D), all_codes, all_codes[..., 0],
            vq_loss, semantic_loss)
